# baseline (device time: 506284 ns/iter reference)
import jax
import jax.numpy as jnp
from jax import lax
from jax.experimental import pallas as pl
from jax.experimental.pallas import tpu as pltpu

N_DEV = 8
B = 2
S = 1024
D = 1024
HPS = 8
DH = 128
SCALE = 0.08838834764831843
EPS = 1e-5
R = B * S
CHUNK = R // N_DEV

_F32 = jnp.float32


def _ln_mod(xb, scale_row, shift_row):
    m = jnp.mean(xb, axis=-1, keepdims=True)
    v = jnp.mean(jnp.square(xb - m), axis=-1, keepdims=True)
    xn = (xb - m) * lax.rsqrt(v + EPS)
    return xn * (1.0 + scale_row)[None, :] + shift_row[None, :]


def _attn_partial(x, Wq, Wk, Wv, Wo, t_emb, W_mod):
    def body(x_ref, wq_ref, wk_ref, wv_ref, wo_ref, temb_ref, wmod_ref,
             partial_ref, mod_ref, xm_ref):
        mod = jnp.dot(temb_ref[:, :], wmod_ref[:, :],
                      preferred_element_type=_F32)
        mod_ref[:, :] = mod
        for b in range(B):
            xm_ref[b] = _ln_mod(x_ref[b], mod[b, 0:D], mod[b, D:2 * D])
            partial_ref[b] = jnp.zeros((S, D), _F32)

        def head_step(h, carry):
            cols = pl.ds(h * DH, DH)
            for b in range(B):
                xm = xm_ref[b]
                q = jnp.dot(xm, wq_ref[:, cols], preferred_element_type=_F32)
                k = jnp.dot(xm, wk_ref[:, cols], preferred_element_type=_F32)
                v = jnp.dot(xm, wv_ref[:, cols], preferred_element_type=_F32)
                s = jnp.dot(q, k.T, preferred_element_type=_F32) * SCALE
                m = jnp.max(s, axis=-1, keepdims=True)
                p = jnp.exp(s - m)
                o = jnp.dot(p, v, preferred_element_type=_F32)
                o = o / jnp.sum(p, axis=-1, keepdims=True)
                partial_ref[b] = partial_ref[b] + jnp.dot(
                    o, wo_ref[pl.ds(h * DH, DH), :], preferred_element_type=_F32)
            return carry

        lax.fori_loop(0, HPS, head_step, 0)

    return pl.pallas_call(
        body,
        out_shape=[
            jax.ShapeDtypeStruct((B, S, D), _F32),
            jax.ShapeDtypeStruct((B, 6 * D), _F32),
        ],
        in_specs=[pl.BlockSpec(memory_space=pltpu.VMEM)] * 7,
        out_specs=[pl.BlockSpec(memory_space=pltpu.VMEM)] * 2,
        scratch_shapes=[pltpu.VMEM((B, S, D), _F32)],
        compiler_params=pltpu.CompilerParams(
            vmem_limit_bytes=120 * 1024 * 1024),
    )(x, Wq, Wk, Wv, Wo, t_emb, W_mod)


def _ffn_partial(x1, W_ff1, W_ff2, mod):
    def body(x_ref, w1_ref, w2_ref, mod_ref, partial_ref):
        for b in range(B):
            xm = _ln_mod(x_ref[b], mod_ref[b, 3 * D:4 * D],
                         mod_ref[b, 4 * D:5 * D])
            h = jnp.dot(xm, w1_ref[:, :], preferred_element_type=_F32)
            h = h / (1.0 + jnp.exp(-h))
            partial_ref[b] = jnp.dot(h, w2_ref[:, :],
                                     preferred_element_type=_F32)

    return pl.pallas_call(
        body,
        out_shape=jax.ShapeDtypeStruct((B, S, D), _F32),
        in_specs=[pl.BlockSpec(memory_space=pltpu.VMEM)] * 4,
        out_specs=pl.BlockSpec(memory_space=pltpu.VMEM),
        compiler_params=pltpu.CompilerParams(
            vmem_limit_bytes=120 * 1024 * 1024),
    )(x1, W_ff1, W_ff2, mod)


def _allreduce_residual(partial2d, base2d, gate, cid):
    def body(p_ref, base_ref, gate_ref, out_ref,
             acc_ref, rs_recv_ref, gather_ref,
             rs_send_sems, rs_recv_sems, ag_send_sems, ag_recv_sems):
        my = lax.axis_index("i")
        left = lax.rem(my + N_DEV - 1, N_DEV)
        right = lax.rem(my + 1, N_DEV)

        barrier = pltpu.get_barrier_semaphore()
        pl.semaphore_signal(barrier, inc=1, device_id=(left,),
                            device_id_type=pl.DeviceIdType.MESH)
        pl.semaphore_signal(barrier, inc=1, device_id=(right,),
                            device_id_type=pl.DeviceIdType.MESH)
        pl.semaphore_wait(barrier, 2)

        acc_ref[:, :] = p_ref[:, :]

        def store_out(c):
            bidx = lax.div(c, S // CHUNK)
            g = gate_ref[pl.ds(bidx, 1), :]
            rows = pl.ds(c * CHUNK, CHUNK)
            out_ref[rows, :] = base_ref[rows, :] + g * gather_ref[rows, :]

        for s in range(N_DEV - 1):
            send_c = lax.rem(my + N_DEV - s, N_DEV)
            recv_c = lax.rem(my + 2 * N_DEV - s - 1, N_DEV)
            rdma = pltpu.make_async_remote_copy(
                src_ref=acc_ref.at[pl.ds(send_c * CHUNK, CHUNK), :],
                dst_ref=rs_recv_ref.at[s],
                send_sem=rs_send_sems.at[s],
                recv_sem=rs_recv_sems.at[s],
                device_id=(right,),
                device_id_type=pl.DeviceIdType.MESH,
            )
            rdma.start()
            rdma.wait()
            rows = pl.ds(recv_c * CHUNK, CHUNK)
            acc_ref[rows, :] = acc_ref[rows, :] + rs_recv_ref[s]

        c0 = lax.rem(my + 1, N_DEV)
        rows0 = pl.ds(c0 * CHUNK, CHUNK)
        gather_ref[rows0, :] = acc_ref[rows0, :]
        store_out(c0)
        for s in range(N_DEV - 1):
            send_c = lax.rem(my + 1 + N_DEV - s, N_DEV)
            recv_c = lax.rem(my + N_DEV - s, N_DEV)
            send_rows = pl.ds(send_c * CHUNK, CHUNK)
            rdma = pltpu.make_async_remote_copy(
                src_ref=gather_ref.at[send_rows, :],
                dst_ref=gather_ref.at[send_rows, :],
                send_sem=ag_send_sems.at[s],
                recv_sem=ag_recv_sems.at[s],
                device_id=(right,),
                device_id_type=pl.DeviceIdType.MESH,
            )
            rdma.start()
            rdma.wait()
            store_out(recv_c)

    return pl.pallas_call(
        body,
        out_shape=jax.ShapeDtypeStruct((R, D), _F32),
        in_specs=[pl.BlockSpec(memory_space=pltpu.VMEM)] * 3,
        out_specs=pl.BlockSpec(memory_space=pltpu.VMEM),
        scratch_shapes=[
            pltpu.VMEM((R, D), _F32),
            pltpu.VMEM((N_DEV - 1, CHUNK, D), _F32),
            pltpu.VMEM((R, D), _F32),
            pltpu.SemaphoreType.DMA((N_DEV - 1,)),
            pltpu.SemaphoreType.DMA((N_DEV - 1,)),
            pltpu.SemaphoreType.DMA((N_DEV - 1,)),
            pltpu.SemaphoreType.DMA((N_DEV - 1,)),
        ],
        compiler_params=pltpu.CompilerParams(
            collective_id=cid,
            vmem_limit_bytes=120 * 1024 * 1024,
        ),
    )(partial2d, base2d, gate)


def kernel(x, Wq, Wk, Wv, Wo, t_emb, W_mod, W_ff1, W_ff2):
    partial_attn, mod = _attn_partial(x, Wq, Wk, Wv, Wo, t_emb, W_mod)
    ga = mod[:, 2 * D:3 * D]
    gm = mod[:, 5 * D:6 * D]
    x1 = _allreduce_residual(
        partial_attn.reshape(R, D), x.reshape(R, D), ga, cid=0)
    x1 = x1.reshape(B, S, D)
    partial_ffn = _ffn_partial(x1, W_ff1, W_ff2, mod)
    out = _allreduce_residual(
        partial_ffn.reshape(R, D), x1.reshape(R, D), gm, cid=1)
    return out.reshape(B, S, D)


# device time: 358678 ns/iter; 1.4115x vs baseline; 1.4115x over previous
import jax
import jax.numpy as jnp
from jax import lax
from jax.experimental import pallas as pl
from jax.experimental.pallas import tpu as pltpu

N_DEV = 8
B = 2
S = 1024
D = 1024
HPS = 8
DH = 128
SCALE = 0.08838834764831843
EPS = 1e-5
R = B * S
CHUNK = R // N_DEV

_F32 = jnp.float32


def _ln_mod(xb, scale_row, shift_row):
    m = jnp.mean(xb, axis=-1, keepdims=True)
    v = jnp.mean(jnp.square(xb - m), axis=-1, keepdims=True)
    xn = (xb - m) * lax.rsqrt(v + EPS)
    return xn * (1.0 + scale_row)[None, :] + shift_row[None, :]


def _attn_partial(x, Wq, Wk, Wv, Wo, t_emb, W_mod):
    def body(x_ref, wq_ref, wk_ref, wv_ref, wo_ref, temb_ref, wmod_ref,
             partial_ref, mod_ref, xm_ref):
        mod = jnp.dot(temb_ref[:, :], wmod_ref[:, :],
                      preferred_element_type=_F32)
        mod_ref[:, :] = mod
        for b in range(B):
            xm_ref[b] = _ln_mod(x_ref[b], mod[b, 0:D], mod[b, D:2 * D])
            partial_ref[b] = jnp.zeros((S, D), _F32)

        def head_step(h, carry):
            cols = pl.ds(h * DH, DH)
            for b in range(B):
                xm = xm_ref[b]
                q = jnp.dot(xm, wq_ref[:, cols], preferred_element_type=_F32)
                k = jnp.dot(xm, wk_ref[:, cols], preferred_element_type=_F32)
                v = jnp.dot(xm, wv_ref[:, cols], preferred_element_type=_F32)
                s = jnp.dot(q, k.T, preferred_element_type=_F32) * SCALE
                m = jnp.max(s, axis=-1, keepdims=True)
                p = jnp.exp(s - m)
                o = jnp.dot(p, v, preferred_element_type=_F32)
                o = o / jnp.sum(p, axis=-1, keepdims=True)
                partial_ref[b] = partial_ref[b] + jnp.dot(
                    o, wo_ref[pl.ds(h * DH, DH), :], preferred_element_type=_F32)
            return carry

        lax.fori_loop(0, HPS, head_step, 0)

    return pl.pallas_call(
        body,
        out_shape=[
            jax.ShapeDtypeStruct((B, S, D), _F32),
            jax.ShapeDtypeStruct((B, 6 * D), _F32),
        ],
        in_specs=[pl.BlockSpec(memory_space=pltpu.VMEM)] * 7,
        out_specs=[pl.BlockSpec(memory_space=pltpu.VMEM)] * 2,
        scratch_shapes=[pltpu.VMEM((B, S, D), _F32)],
        compiler_params=pltpu.CompilerParams(
            vmem_limit_bytes=120 * 1024 * 1024),
    )(x, Wq, Wk, Wv, Wo, t_emb, W_mod)


def _ffn_partial(x1, W_ff1, W_ff2, mod):
    def body(x_ref, w1_ref, w2_ref, mod_ref, partial_ref):
        for b in range(B):
            xm = _ln_mod(x_ref[b], mod_ref[b, 3 * D:4 * D],
                         mod_ref[b, 4 * D:5 * D])
            h = jnp.dot(xm, w1_ref[:, :], preferred_element_type=_F32)
            h = h / (1.0 + jnp.exp(-h))
            partial_ref[b] = jnp.dot(h, w2_ref[:, :],
                                     preferred_element_type=_F32)

    return pl.pallas_call(
        body,
        out_shape=jax.ShapeDtypeStruct((B, S, D), _F32),
        in_specs=[pl.BlockSpec(memory_space=pltpu.VMEM)] * 4,
        out_specs=pl.BlockSpec(memory_space=pltpu.VMEM),
        compiler_params=pltpu.CompilerParams(
            vmem_limit_bytes=120 * 1024 * 1024),
    )(x1, W_ff1, W_ff2, mod)


HALF = D // 2


def _allreduce_residual(partial2d, base2d, gate, cid):
    def body(p_ref, base_ref, gate_ref, out_ref,
             acc_ref, rs_recv_cw, rs_recv_ccw, gather_ref,
             rs_send_sems_cw, rs_recv_sems_cw,
             rs_send_sems_ccw, rs_recv_sems_ccw,
             ag_send_sems_cw, ag_recv_sems_cw,
             ag_send_sems_ccw, ag_recv_sems_ccw):
        my = lax.axis_index("i")
        left = lax.rem(my + N_DEV - 1, N_DEV)
        right = lax.rem(my + 1, N_DEV)
        cw_cols = pl.ds(0, HALF)
        ccw_cols = pl.ds(HALF, HALF)

        barrier = pltpu.get_barrier_semaphore()
        pl.semaphore_signal(barrier, inc=1, device_id=(left,),
                            device_id_type=pl.DeviceIdType.MESH)
        pl.semaphore_signal(barrier, inc=1, device_id=(right,),
                            device_id_type=pl.DeviceIdType.MESH)
        pl.semaphore_wait(barrier, 2)

        acc_ref[:, :] = p_ref[:, :]

        def store_out(c, cols):
            bidx = lax.div(c, S // CHUNK)
            g = gate_ref[pl.ds(bidx, 1), cols]
            rows = pl.ds(c * CHUNK, CHUNK)
            out_ref[rows, cols] = base_ref[rows, cols] + g * gather_ref[rows, cols]

        for s in range(N_DEV - 1):
            send_cw = lax.rem(my + N_DEV - s, N_DEV)
            recv_cw = lax.rem(my + 2 * N_DEV - s - 1, N_DEV)
            send_ccw = lax.rem(my + s, N_DEV)
            recv_ccw = lax.rem(my + s + 1, N_DEV)
            rdma_cw = pltpu.make_async_remote_copy(
                src_ref=acc_ref.at[pl.ds(send_cw * CHUNK, CHUNK), cw_cols],
                dst_ref=rs_recv_cw.at[s],
                send_sem=rs_send_sems_cw.at[s],
                recv_sem=rs_recv_sems_cw.at[s],
                device_id=(right,),
                device_id_type=pl.DeviceIdType.MESH,
            )
            rdma_ccw = pltpu.make_async_remote_copy(
                src_ref=acc_ref.at[pl.ds(send_ccw * CHUNK, CHUNK), ccw_cols],
                dst_ref=rs_recv_ccw.at[s],
                send_sem=rs_send_sems_ccw.at[s],
                recv_sem=rs_recv_sems_ccw.at[s],
                device_id=(left,),
                device_id_type=pl.DeviceIdType.MESH,
            )
            rdma_cw.start()
            rdma_ccw.start()
            rdma_cw.wait()
            rdma_ccw.wait()
            rows_cw = pl.ds(recv_cw * CHUNK, CHUNK)
            acc_ref[rows_cw, cw_cols] = acc_ref[rows_cw, cw_cols] + rs_recv_cw[s]
            rows_ccw = pl.ds(recv_ccw * CHUNK, CHUNK)
            acc_ref[rows_ccw, ccw_cols] = (
                acc_ref[rows_ccw, ccw_cols] + rs_recv_ccw[s])

        c0 = lax.rem(my + 1, N_DEV)
        rows0 = pl.ds(c0 * CHUNK, CHUNK)
        gather_ref[rows0, cw_cols] = acc_ref[rows0, cw_cols]
        store_out(c0, cw_cols)
        c1 = lax.rem(my + N_DEV - 1, N_DEV)
        rows1 = pl.ds(c1 * CHUNK, CHUNK)
        gather_ref[rows1, ccw_cols] = acc_ref[rows1, ccw_cols]
        store_out(c1, ccw_cols)
        for s in range(N_DEV - 1):
            send_cw = lax.rem(my + 1 + N_DEV - s, N_DEV)
            recv_cw = lax.rem(my + N_DEV - s, N_DEV)
            send_ccw = lax.rem(my + N_DEV - 1 + s, N_DEV)
            recv_ccw = lax.rem(my + s, N_DEV)
            rows_s_cw = pl.ds(send_cw * CHUNK, CHUNK)
            rows_s_ccw = pl.ds(send_ccw * CHUNK, CHUNK)
            rdma_cw = pltpu.make_async_remote_copy(
                src_ref=gather_ref.at[rows_s_cw, cw_cols],
                dst_ref=gather_ref.at[rows_s_cw, cw_cols],
                send_sem=ag_send_sems_cw.at[s],
                recv_sem=ag_recv_sems_cw.at[s],
                device_id=(right,),
                device_id_type=pl.DeviceIdType.MESH,
            )
            rdma_ccw = pltpu.make_async_remote_copy(
                src_ref=gather_ref.at[rows_s_ccw, ccw_cols],
                dst_ref=gather_ref.at[rows_s_ccw, ccw_cols],
                send_sem=ag_send_sems_ccw.at[s],
                recv_sem=ag_recv_sems_ccw.at[s],
                device_id=(left,),
                device_id_type=pl.DeviceIdType.MESH,
            )
            rdma_cw.start()
            rdma_ccw.start()
            rdma_cw.wait()
            rdma_ccw.wait()
            store_out(recv_cw, cw_cols)
            store_out(recv_ccw, ccw_cols)

    dma7 = pltpu.SemaphoreType.DMA((N_DEV - 1,))
    return pl.pallas_call(
        body,
        out_shape=jax.ShapeDtypeStruct((R, D), _F32),
        in_specs=[pl.BlockSpec(memory_space=pltpu.VMEM)] * 3,
        out_specs=pl.BlockSpec(memory_space=pltpu.VMEM),
        scratch_shapes=[
            pltpu.VMEM((R, D), _F32),
            pltpu.VMEM((N_DEV - 1, CHUNK, HALF), _F32),
            pltpu.VMEM((N_DEV - 1, CHUNK, HALF), _F32),
            pltpu.VMEM((R, D), _F32),
            dma7, dma7, dma7, dma7, dma7, dma7, dma7, dma7,
        ],
        compiler_params=pltpu.CompilerParams(
            collective_id=cid,
            vmem_limit_bytes=120 * 1024 * 1024,
        ),
    )(partial2d, base2d, gate)


def kernel(x, Wq, Wk, Wv, Wo, t_emb, W_mod, W_ff1, W_ff2):
    partial_attn, mod = _attn_partial(x, Wq, Wk, Wv, Wo, t_emb, W_mod)
    ga = mod[:, 2 * D:3 * D]
    gm = mod[:, 5 * D:6 * D]
    x1 = _allreduce_residual(
        partial_attn.reshape(R, D), x.reshape(R, D), ga, cid=0)
    x1 = x1.reshape(B, S, D)
    partial_ffn = _ffn_partial(x1, W_ff1, W_ff2, mod)
    out = _allreduce_residual(
        partial_ffn.reshape(R, D), x1.reshape(R, D), gm, cid=1)
    return out.reshape(B, S, D)


# device time: 281392 ns/iter; 1.7992x vs baseline; 1.2747x over previous
import jax
import jax.numpy as jnp
from jax import lax
from jax.experimental import pallas as pl
from jax.experimental.pallas import tpu as pltpu

N_DEV = 8
B = 2
S = 1024
D = 1024
HPS = 8
DH = 128
SCALE = 0.08838834764831843
EPS = 1e-5
R = B * S
CHUNK = R // N_DEV

_F32 = jnp.float32


def _ln_mod(xb, scale_row, shift_row):
    m = jnp.mean(xb, axis=-1, keepdims=True)
    v = jnp.mean(jnp.square(xb - m), axis=-1, keepdims=True)
    xn = (xb - m) * lax.rsqrt(v + EPS)
    return xn * (1.0 + scale_row)[None, :] + shift_row[None, :]


def _attn_partial(x, Wq, Wk, Wv, Wo, t_emb, W_mod):
    def body(x_ref, wq_ref, wk_ref, wv_ref, wo_ref, temb_ref, wmod_ref,
             partial_ref, mod_ref, xm_ref):
        mod = jnp.dot(temb_ref[:, :], wmod_ref[:, :],
                      preferred_element_type=_F32)
        mod_ref[:, :] = mod
        for b in range(B):
            xm_ref[b] = _ln_mod(x_ref[b], mod[b, 0:D], mod[b, D:2 * D])
            partial_ref[b] = jnp.zeros((S, D), _F32)

        def head_step(h, carry):
            cols = pl.ds(h * DH, DH)
            for b in range(B):
                xm = xm_ref[b]
                q = jnp.dot(xm, wq_ref[:, cols], preferred_element_type=_F32)
                k = jnp.dot(xm, wk_ref[:, cols], preferred_element_type=_F32)
                v = jnp.dot(xm, wv_ref[:, cols], preferred_element_type=_F32)
                s = jnp.dot(q, k.T, preferred_element_type=_F32) * SCALE
                m = jnp.max(s, axis=-1, keepdims=True)
                p = jnp.exp(s - m)
                o = jnp.dot(p, v, preferred_element_type=_F32)
                o = o / jnp.sum(p, axis=-1, keepdims=True)
                partial_ref[b] = partial_ref[b] + jnp.dot(
                    o, wo_ref[pl.ds(h * DH, DH), :], preferred_element_type=_F32)
            return carry

        lax.fori_loop(0, HPS, head_step, 0)

    return pl.pallas_call(
        body,
        out_shape=[
            jax.ShapeDtypeStruct((B, S, D), _F32),
            jax.ShapeDtypeStruct((B, 6 * D), _F32),
        ],
        in_specs=[pl.BlockSpec(memory_space=pltpu.VMEM)] * 7,
        out_specs=[pl.BlockSpec(memory_space=pltpu.VMEM)] * 2,
        scratch_shapes=[pltpu.VMEM((B, S, D), _F32)],
        compiler_params=pltpu.CompilerParams(
            vmem_limit_bytes=120 * 1024 * 1024),
    )(x, Wq, Wk, Wv, Wo, t_emb, W_mod)


def _ffn_partial(x1, W_ff1, W_ff2, mod):
    def body(x_ref, w1_ref, w2_ref, mod_ref, partial_ref):
        for b in range(B):
            xm = _ln_mod(x_ref[b], mod_ref[b, 3 * D:4 * D],
                         mod_ref[b, 4 * D:5 * D])
            h = jnp.dot(xm, w1_ref[:, :], preferred_element_type=_F32)
            h = h / (1.0 + jnp.exp(-h))
            partial_ref[b] = jnp.dot(h, w2_ref[:, :],
                                     preferred_element_type=_F32)

    return pl.pallas_call(
        body,
        out_shape=jax.ShapeDtypeStruct((B, S, D), _F32),
        in_specs=[pl.BlockSpec(memory_space=pltpu.VMEM)] * 4,
        out_specs=pl.BlockSpec(memory_space=pltpu.VMEM),
        compiler_params=pltpu.CompilerParams(
            vmem_limit_bytes=120 * 1024 * 1024),
    )(x1, W_ff1, W_ff2, mod)


HALF = D // 2
_BF16 = jnp.bfloat16


def _allreduce_residual(partial2d, base2d, gate, cid):
    def body(p_ref, base_ref, gate_ref, out_ref,
             acc_ref, rs_send_cw, rs_send_ccw, rs_recv_cw, rs_recv_ccw,
             gather_ref,
             rs_send_sems_cw, rs_recv_sems_cw,
             rs_send_sems_ccw, rs_recv_sems_ccw,
             ag_send_sems_cw, ag_recv_sems_cw,
             ag_send_sems_ccw, ag_recv_sems_ccw):
        my = lax.axis_index("i")
        left = lax.rem(my + N_DEV - 1, N_DEV)
        right = lax.rem(my + 1, N_DEV)
        cw_cols = pl.ds(0, HALF)
        ccw_cols = pl.ds(HALF, HALF)

        barrier = pltpu.get_barrier_semaphore()
        pl.semaphore_signal(barrier, inc=1, device_id=(left,),
                            device_id_type=pl.DeviceIdType.MESH)
        pl.semaphore_signal(barrier, inc=1, device_id=(right,),
                            device_id_type=pl.DeviceIdType.MESH)
        pl.semaphore_wait(barrier, 2)

        acc_ref[:, :] = p_ref[:, :]

        def store_out(c, cols):
            bidx = lax.div(c, S // CHUNK)
            g = gate_ref[pl.ds(bidx, 1), cols]
            rows = pl.ds(c * CHUNK, CHUNK)
            out_ref[rows, cols] = base_ref[rows, cols] + g * (
                gather_ref[rows, cols].astype(_F32))

        for s in range(N_DEV - 1):
            send_cw = lax.rem(my + N_DEV - s, N_DEV)
            recv_cw = lax.rem(my + 2 * N_DEV - s - 1, N_DEV)
            send_ccw = lax.rem(my + s, N_DEV)
            recv_ccw = lax.rem(my + s + 1, N_DEV)
            rs_send_cw[s] = acc_ref[pl.ds(send_cw * CHUNK, CHUNK),
                                    cw_cols].astype(_BF16)
            rs_send_ccw[s] = acc_ref[pl.ds(send_ccw * CHUNK, CHUNK),
                                     ccw_cols].astype(_BF16)
            rdma_cw = pltpu.make_async_remote_copy(
                src_ref=rs_send_cw.at[s],
                dst_ref=rs_recv_cw.at[s],
                send_sem=rs_send_sems_cw.at[s],
                recv_sem=rs_recv_sems_cw.at[s],
                device_id=(right,),
                device_id_type=pl.DeviceIdType.MESH,
            )
            rdma_ccw = pltpu.make_async_remote_copy(
                src_ref=rs_send_ccw.at[s],
                dst_ref=rs_recv_ccw.at[s],
                send_sem=rs_send_sems_ccw.at[s],
                recv_sem=rs_recv_sems_ccw.at[s],
                device_id=(left,),
                device_id_type=pl.DeviceIdType.MESH,
            )
            rdma_cw.start()
            rdma_ccw.start()
            rdma_cw.wait()
            rdma_ccw.wait()
            rows_cw = pl.ds(recv_cw * CHUNK, CHUNK)
            acc_ref[rows_cw, cw_cols] = (
                acc_ref[rows_cw, cw_cols] + rs_recv_cw[s].astype(_F32))
            rows_ccw = pl.ds(recv_ccw * CHUNK, CHUNK)
            acc_ref[rows_ccw, ccw_cols] = (
                acc_ref[rows_ccw, ccw_cols] + rs_recv_ccw[s].astype(_F32))

        c0 = lax.rem(my + 1, N_DEV)
        rows0 = pl.ds(c0 * CHUNK, CHUNK)
        gather_ref[rows0, cw_cols] = acc_ref[rows0, cw_cols].astype(_BF16)
        store_out(c0, cw_cols)
        c1 = lax.rem(my + N_DEV - 1, N_DEV)
        rows1 = pl.ds(c1 * CHUNK, CHUNK)
        gather_ref[rows1, ccw_cols] = acc_ref[rows1, ccw_cols].astype(_BF16)
        store_out(c1, ccw_cols)
        for s in range(N_DEV - 1):
            send_cw = lax.rem(my + 1 + N_DEV - s, N_DEV)
            recv_cw = lax.rem(my + N_DEV - s, N_DEV)
            send_ccw = lax.rem(my + N_DEV - 1 + s, N_DEV)
            recv_ccw = lax.rem(my + s, N_DEV)
            rows_s_cw = pl.ds(send_cw * CHUNK, CHUNK)
            rows_s_ccw = pl.ds(send_ccw * CHUNK, CHUNK)
            rdma_cw = pltpu.make_async_remote_copy(
                src_ref=gather_ref.at[rows_s_cw, cw_cols],
                dst_ref=gather_ref.at[rows_s_cw, cw_cols],
                send_sem=ag_send_sems_cw.at[s],
                recv_sem=ag_recv_sems_cw.at[s],
                device_id=(right,),
                device_id_type=pl.DeviceIdType.MESH,
            )
            rdma_ccw = pltpu.make_async_remote_copy(
                src_ref=gather_ref.at[rows_s_ccw, ccw_cols],
                dst_ref=gather_ref.at[rows_s_ccw, ccw_cols],
                send_sem=ag_send_sems_ccw.at[s],
                recv_sem=ag_recv_sems_ccw.at[s],
                device_id=(left,),
                device_id_type=pl.DeviceIdType.MESH,
            )
            rdma_cw.start()
            rdma_ccw.start()
            rdma_cw.wait()
            rdma_ccw.wait()
            store_out(recv_cw, cw_cols)
            store_out(recv_ccw, ccw_cols)

    dma7 = pltpu.SemaphoreType.DMA((N_DEV - 1,))
    return pl.pallas_call(
        body,
        out_shape=jax.ShapeDtypeStruct((R, D), _F32),
        in_specs=[pl.BlockSpec(memory_space=pltpu.VMEM)] * 3,
        out_specs=pl.BlockSpec(memory_space=pltpu.VMEM),
        scratch_shapes=[
            pltpu.VMEM((R, D), _F32),
            pltpu.VMEM((N_DEV - 1, CHUNK, HALF), _BF16),
            pltpu.VMEM((N_DEV - 1, CHUNK, HALF), _BF16),
            pltpu.VMEM((N_DEV - 1, CHUNK, HALF), _BF16),
            pltpu.VMEM((N_DEV - 1, CHUNK, HALF), _BF16),
            pltpu.VMEM((R, D), _BF16),
            dma7, dma7, dma7, dma7, dma7, dma7, dma7, dma7,
        ],
        compiler_params=pltpu.CompilerParams(
            collective_id=cid,
            vmem_limit_bytes=120 * 1024 * 1024,
        ),
    )(partial2d, base2d, gate)


def kernel(x, Wq, Wk, Wv, Wo, t_emb, W_mod, W_ff1, W_ff2):
    partial_attn, mod = _attn_partial(x, Wq, Wk, Wv, Wo, t_emb, W_mod)
    ga = mod[:, 2 * D:3 * D]
    gm = mod[:, 5 * D:6 * D]
    x1 = _allreduce_residual(
        partial_attn.reshape(R, D), x.reshape(R, D), ga, cid=0)
    x1 = x1.reshape(B, S, D)
    partial_ffn = _ffn_partial(x1, W_ff1, W_ff2, mod)
    out = _allreduce_residual(
        partial_ffn.reshape(R, D), x1.reshape(R, D), gm, cid=1)
    return out.reshape(B, S, D)


# device time: 267551 ns/iter; 1.8923x vs baseline; 1.0517x over previous
import jax
import jax.numpy as jnp
from jax import lax
from jax.experimental import pallas as pl
from jax.experimental.pallas import tpu as pltpu

N_DEV = 8
B = 2
S = 1024
D = 1024
HPS = 8
DH = 128
SCALE = 0.08838834764831843
EPS = 1e-5
R = B * S
CHUNK = R // N_DEV

_F32 = jnp.float32
_BF16 = jnp.bfloat16


def _ln_mod(xb, scale_row, shift_row):
    m = jnp.mean(xb, axis=-1, keepdims=True)
    v = jnp.mean(jnp.square(xb - m), axis=-1, keepdims=True)
    xn = (xb - m) * lax.rsqrt(v + EPS)
    return xn * (1.0 + scale_row)[None, :] + shift_row[None, :]


def _attn_partial(x, Wq, Wk, Wv, Wo, t_emb, W_mod):
    def body(x_ref, wq_ref, wk_ref, wv_ref, wo_ref, temb_ref, wmod_ref,
             partial_ref, mod_ref, xm_ref):
        mod = jnp.dot(temb_ref[:, :], wmod_ref[:, :],
                      preferred_element_type=_F32)
        mod_ref[:, :] = mod
        for b in range(B):
            xm_ref[b] = _ln_mod(x_ref[b], mod[b, 0:D], mod[b, D:2 * D])
            partial_ref[b] = jnp.zeros((S, D), _F32)

        def head_step(h, carry):
            cols = pl.ds(h * DH, DH)
            for b in range(B):
                xm = xm_ref[b]
                q = jnp.dot(xm, wq_ref[:, cols], preferred_element_type=_F32)
                k = jnp.dot(xm, wk_ref[:, cols], preferred_element_type=_F32)
                v = jnp.dot(xm, wv_ref[:, cols], preferred_element_type=_F32)
                s = jnp.dot(q, k.T, preferred_element_type=_F32) * SCALE
                m = jnp.max(s, axis=-1, keepdims=True)
                p = jnp.exp(s - m)
                o = jnp.dot(p, v, preferred_element_type=_F32)
                o = o / jnp.sum(p, axis=-1, keepdims=True)
                partial_ref[b] = partial_ref[b] + jnp.dot(
                    o, wo_ref[pl.ds(h * DH, DH), :], preferred_element_type=_F32)
            return carry

        lax.fori_loop(0, HPS, head_step, 0)

    return pl.pallas_call(
        body,
        out_shape=[
            jax.ShapeDtypeStruct((B, S, D), _F32),
            jax.ShapeDtypeStruct((B, 6 * D), _F32),
        ],
        in_specs=[pl.BlockSpec(memory_space=pltpu.VMEM)] * 7,
        out_specs=[pl.BlockSpec(memory_space=pltpu.VMEM)] * 2,
        scratch_shapes=[pltpu.VMEM((B, S, D), _F32)],
        compiler_params=pltpu.CompilerParams(
            vmem_limit_bytes=120 * 1024 * 1024),
    )(x, Wq, Wk, Wv, Wo, t_emb, W_mod)


def _ffn_partial(x1, W_ff1, W_ff2, mod):
    def body(x_ref, w1_ref, w2_ref, mod_ref, partial_ref):
        for b in range(B):
            xm = _ln_mod(x_ref[b], mod_ref[b, 3 * D:4 * D],
                         mod_ref[b, 4 * D:5 * D])
            h = jnp.dot(xm, w1_ref[:, :], preferred_element_type=_F32)
            h = h / (1.0 + jnp.exp(-h))
            partial_ref[b] = jnp.dot(h, w2_ref[:, :],
                                     preferred_element_type=_F32)

    return pl.pallas_call(
        body,
        out_shape=jax.ShapeDtypeStruct((B, S, D), _F32),
        in_specs=[pl.BlockSpec(memory_space=pltpu.VMEM)] * 4,
        out_specs=pl.BlockSpec(memory_space=pltpu.VMEM),
        compiler_params=pltpu.CompilerParams(
            vmem_limit_bytes=120 * 1024 * 1024),
    )(x1, W_ff1, W_ff2, mod)


def _allreduce_residual(partial2d, base2d, gate, cid):
    def body(p_ref, base_ref, gate_ref, out_ref,
             rs_send, rs_recv, ag_send, g_ref,
             rs_send_sems, rs_recv_sems, ag_send_sems, ag_recv_sems):
        my = lax.axis_index("i")

        barrier = pltpu.get_barrier_semaphore()
        for j in range(1, N_DEV):
            peer = lax.rem(my + j, N_DEV)
            pl.semaphore_signal(barrier, inc=1, device_id=(peer,),
                                device_id_type=pl.DeviceIdType.MESH)
        pl.semaphore_wait(barrier, N_DEV - 1)

        rs_descs = []
        for j in range(1, N_DEV):
            t = lax.rem(my + j, N_DEV)
            rs_send[j - 1] = p_ref[pl.ds(t * CHUNK, CHUNK), :].astype(_BF16)
            rdma = pltpu.make_async_remote_copy(
                src_ref=rs_send.at[j - 1],
                dst_ref=rs_recv.at[j - 1],
                send_sem=rs_send_sems.at[j - 1],
                recv_sem=rs_recv_sems.at[j - 1],
                device_id=(t,),
                device_id_type=pl.DeviceIdType.MESH,
            )
            rdma.start()
            rs_descs.append(rdma)

        own_rows = pl.ds(my * CHUNK, CHUNK)
        acc = p_ref[own_rows, :]
        for j in range(1, N_DEV):
            rs_descs[j - 1].wait_recv()
            acc = acc + rs_recv[j - 1].astype(_F32)

        def gate_row(c):
            bidx = lax.div(c, S // CHUNK)
            return gate_ref[pl.ds(bidx, 1), :]

        out_ref[own_rows, :] = base_ref[own_rows, :] + gate_row(my) * acc
        ag_send[:, :] = acc.astype(_BF16)

        ag_descs = []
        for j in range(1, N_DEV):
            t = lax.rem(my + j, N_DEV)
            rdma = pltpu.make_async_remote_copy(
                src_ref=ag_send,
                dst_ref=g_ref.at[own_rows, :],
                send_sem=ag_send_sems.at[j - 1],
                recv_sem=ag_recv_sems.at[my],
                device_id=(t,),
                device_id_type=pl.DeviceIdType.MESH,
            )
            rdma.start()
            ag_descs.append(rdma)

        for j in range(1, N_DEV):
            t = lax.rem(my + j, N_DEV)
            rows = pl.ds(t * CHUNK, CHUNK)
            recv = pltpu.make_async_remote_copy(
                src_ref=ag_send,
                dst_ref=g_ref.at[rows, :],
                send_sem=ag_send_sems.at[j - 1],
                recv_sem=ag_recv_sems.at[t],
                device_id=(t,),
                device_id_type=pl.DeviceIdType.MESH,
            )
            recv.wait_recv()
            out_ref[rows, :] = base_ref[rows, :] + gate_row(t) * (
                g_ref[rows, :].astype(_F32))

        for d in rs_descs:
            d.wait_send()
        for d in ag_descs:
            d.wait_send()

    dma7 = pltpu.SemaphoreType.DMA((N_DEV - 1,))
    dma8 = pltpu.SemaphoreType.DMA((N_DEV,))
    return pl.pallas_call(
        body,
        out_shape=jax.ShapeDtypeStruct((R, D), _F32),
        in_specs=[pl.BlockSpec(memory_space=pltpu.VMEM)] * 3,
        out_specs=pl.BlockSpec(memory_space=pltpu.VMEM),
        scratch_shapes=[
            pltpu.VMEM((N_DEV - 1, CHUNK, D), _BF16),
            pltpu.VMEM((N_DEV - 1, CHUNK, D), _BF16),
            pltpu.VMEM((CHUNK, D), _BF16),
            pltpu.VMEM((R, D), _BF16),
            dma7, dma7, dma7, dma8,
        ],
        compiler_params=pltpu.CompilerParams(
            collective_id=cid,
            vmem_limit_bytes=120 * 1024 * 1024,
        ),
    )(partial2d, base2d, gate)


def kernel(x, Wq, Wk, Wv, Wo, t_emb, W_mod, W_ff1, W_ff2):
    partial_attn, mod = _attn_partial(x, Wq, Wk, Wv, Wo, t_emb, W_mod)
    ga = mod[:, 2 * D:3 * D]
    gm = mod[:, 5 * D:6 * D]
    x1 = _allreduce_residual(
        partial_attn.reshape(R, D), x.reshape(R, D), ga, cid=0)
    x1 = x1.reshape(B, S, D)
    partial_ffn = _ffn_partial(x1, W_ff1, W_ff2, mod)
    out = _allreduce_residual(
        partial_ffn.reshape(R, D), x1.reshape(R, D), gm, cid=1)
    return out.reshape(B, S, D)
